# baseline (speedup 1.0000x reference)
import functools

import jax
import jax.numpy as jnp
from jax import lax
from jax.experimental import pallas as pl
from jax.experimental.pallas import tpu as pltpu
from jax.experimental.pallas import tpu_sc as plsc

NUSERS = 20000
NN = 50000
DD = 64
EE = 800000
NC, NS, LANES = 2, 16, 16
NW = NC * NS
HALF = NN // 2
QN = NN // 4
TRASH = QN
ACC_STRIPE = 784
ACC_ROWS = NS * ACC_STRIPE
DEG_WORDS = 16 * 3128

CH = 128
CHA = 1024
QCAP = 26624
FCH = 2048
PCH = 512
PER_W = EE // NW
NPF = PER_W // PCH
PTAIL = PER_W - NPF * PCH

_mesh = plsc.VectorSubcoreMesh(
    core_axis_name="c", subcore_axis_name="s", num_cores=NC, num_subcores=NS)
_sc_params = pltpu.CompilerParams(use_tc_tiling_on_sc=False,
                                  needs_layout_passes=False)


def _deg_body(dst_hbm, ones_hbm, z_hbm, deg_out, didx, didx_t, ones_v, zv,
              acc):
    c = lax.axis_index("c")
    s = lax.axis_index("s")
    pltpu.sync_copy(z_hbm, zv)
    pltpu.sync_copy(zv, acc.at[pl.ds(pl.multiple_of(s * 3128, 8), 3128), :])
    pltpu.sync_copy(ones_hbm, ones_v)
    plsc.subcore_barrier()

    base = (c * NS + s) * PER_W
    nfull = PER_W // CH
    tail = PER_W - nfull * CH

    def body(i, _):
        off = pl.multiple_of(base + i * CH, 8)
        pltpu.sync_copy(dst_hbm.at[pl.ds(off, CH)], didx)
        pltpu.sync_copy(ones_v, acc.at[didx, :], add=True)
        return ()

    lax.fori_loop(0, nfull, body, (), unroll=False)
    off = pl.multiple_of(base + nfull * CH, 8)
    pltpu.sync_copy(dst_hbm.at[pl.ds(off, tail)], didx_t)
    pltpu.sync_copy(ones_v.at[pl.ds(0, tail), :], acc.at[didx_t, :], add=True)

    plsc.subcore_barrier()
    nw = 3128
    w0 = pl.multiple_of(s * nw, 8)
    last = NN - 15 * nw

    @pl.when(s < NS - 1)
    def _():
        pltpu.sync_copy(acc.at[pl.ds(w0, nw), :],
                        deg_out.at[c, pl.ds(w0, nw), :])

    @pl.when(s == NS - 1)
    def _():
        pltpu.sync_copy(acc.at[pl.ds(w0, last), :],
                        deg_out.at[c, pl.ds(w0, last), :])


_deg_kernel = functools.partial(
    pl.kernel,
    out_type=jax.ShapeDtypeStruct((NC, NN, 16), jnp.float32),
    mesh=_mesh,
    scratch_types=[
        pltpu.VMEM((CH,), jnp.int32),
        pltpu.VMEM((40,), jnp.int32),
        pltpu.VMEM((CH, 16), jnp.float32),
        pltpu.VMEM((3128, 16), jnp.float32),
        pltpu.VMEM_SHARED((DEG_WORDS, 16), jnp.float32),
    ],
    compiler_params=_sc_params,
)(_deg_body)


def _part_body(src_hbm, dst_hbm, ps_out, pd_out, cnt_out,
               sbuf, dbuf, osa, oda, osb, odb, cbuf):
    c = lax.axis_index("c")
    s = lax.axis_index("s")
    t = c * NS + s
    tbase = t * PER_W

    lane = lax.iota(jnp.int32, 16)
    zero_v = jnp.zeros((LANES,), jnp.int32)
    trash_v = jnp.full((LANES,), TRASH, jnp.int32)

    for p in range(2):
        qa = 2 * p
        lo_a, lo_b = qa * QN, (qa + 1) * QN

        def scan_vregs(nv, last_lanes, ca, cb, qa=qa, lo_a=lo_a, lo_b=lo_b):
            for j in range(nv):
                sv = sbuf[pl.ds(j * LANES, LANES)]
                dv = dbuf[pl.ds(j * LANES, LANES)]
                in_a = (dv >= lo_a) & (dv < lo_b)
                in_b = (dv >= lo_b) & (dv < lo_b + QN)
                if j == nv - 1 and last_lanes < LANES:
                    valid = lane < last_lanes
                    in_a = in_a & valid
                    in_b = in_b & valid
                plsc.store_compressed(osa.at[pl.ds(ca, LANES)], sv, mask=in_a)
                plsc.store_compressed(oda.at[pl.ds(ca, LANES)], dv - lo_a,
                                      mask=in_a)
                plsc.store_compressed(osb.at[pl.ds(cb, LANES)], sv, mask=in_b)
                plsc.store_compressed(odb.at[pl.ds(cb, LANES)], dv - lo_b,
                                      mask=in_b)
                ca = ca + jnp.sum(in_a.astype(jnp.int32))
                cb = cb + jnp.sum(in_b.astype(jnp.int32))
            return ca, cb

        def chunk(i, carry, scan_vregs=scan_vregs):
            ca, cb = carry
            off = pl.multiple_of(tbase + i * PCH, 8)
            pltpu.sync_copy(src_hbm.at[pl.ds(off, PCH)], sbuf)
            pltpu.sync_copy(dst_hbm.at[pl.ds(off, PCH)], dbuf)
            return scan_vregs(PCH // LANES, LANES, ca, cb)

        z = jnp.int32(0)
        ca, cb = lax.fori_loop(0, NPF, chunk, (z, z), unroll=False)
        off = pl.multiple_of(tbase + NPF * PCH, 8)
        pltpu.sync_copy(src_hbm.at[pl.ds(off, PTAIL)],
                        sbuf.at[pl.ds(0, PTAIL)])
        pltpu.sync_copy(dst_hbm.at[pl.ds(off, PTAIL)],
                        dbuf.at[pl.ds(0, PTAIL)])
        ca, cb = scan_vregs(PTAIL // LANES + 1, PTAIL % LANES, ca, cb)

        for k in range(CHA // LANES):
            osa[pl.ds(ca + k * LANES, LANES)] = zero_v
            oda[pl.ds(ca + k * LANES, LANES)] = trash_v
            osb[pl.ds(cb + k * LANES, LANES)] = zero_v
            odb[pl.ds(cb + k * LANES, LANES)] = trash_v
        rca = (ca + CHA - 1) // CHA * CHA
        rcb = (cb + CHA - 1) // CHA * CHA

        cbuf[0, :] = jnp.full((LANES,), rca, jnp.int32)
        cbuf[1, :] = jnp.full((LANES,), rcb, jnp.int32)
        pltpu.sync_copy(cbuf.at[0], cnt_out.at[qa, t])
        pltpu.sync_copy(cbuf.at[1], cnt_out.at[qa + 1, t])

        def flush_a(k, _, qa=qa):
            o = pl.multiple_of(k * FCH, 8)
            pltpu.sync_copy(osa.at[pl.ds(o, FCH)],
                            ps_out.at[qa, t, pl.ds(o, FCH)])
            pltpu.sync_copy(oda.at[pl.ds(o, FCH)],
                            pd_out.at[qa, t, pl.ds(o, FCH)])
            return ()

        def flush_b(k, _, qa=qa):
            o = pl.multiple_of(k * FCH, 8)
            pltpu.sync_copy(osb.at[pl.ds(o, FCH)],
                            ps_out.at[qa + 1, t, pl.ds(o, FCH)])
            pltpu.sync_copy(odb.at[pl.ds(o, FCH)],
                            pd_out.at[qa + 1, t, pl.ds(o, FCH)])
            return ()

        lax.fori_loop(0, (rca + FCH - 1) // FCH, flush_a, (), unroll=False)
        lax.fori_loop(0, (rcb + FCH - 1) // FCH, flush_b, (), unroll=False)


_part_kernel = functools.partial(
    pl.kernel,
    out_type=[jax.ShapeDtypeStruct((4, NW, QCAP), jnp.int32),
              jax.ShapeDtypeStruct((4, NW, QCAP), jnp.int32),
              jax.ShapeDtypeStruct((4, NW, 16), jnp.int32)],
    mesh=_mesh,
    scratch_types=[
        pltpu.VMEM((PCH,), jnp.int32),
        pltpu.VMEM((PCH,), jnp.int32),
        pltpu.VMEM((QCAP,), jnp.int32),
        pltpu.VMEM((QCAP,), jnp.int32),
        pltpu.VMEM((QCAP,), jnp.int32),
        pltpu.VMEM((QCAP,), jnp.int32),
        pltpu.VMEM((2, LANES), jnp.int32),
    ],
    compiler_params=_sc_params,
)(_part_body)


def _make_agg(p):
    def _agg_body(g_hbm, ps_hbm, pd_hbm, pc_hbm, z_hbm, out_hbm,
                  sidx0, dbuf0, cbuf, rows0, zv, acc, gsem0):
        c = lax.axis_index("c")
        s = lax.axis_index("s")
        q = 2 * p + c

        r0 = s * ACC_STRIPE
        last = QN - 15 * ACC_STRIPE

        pltpu.sync_copy(z_hbm, zv)
        for k in range(8):
            pltpu.sync_copy(zv, acc.at[pl.ds(r0 + k * 98, 98), :])
        plsc.subcore_barrier()

        for r in range(2):
            t = s * 2 + r
            pltpu.sync_copy(pc_hbm.at[q, t], cbuf)
            rc = lax.reduce_max(cbuf[...], axes=(0,))
            nch = rc // CHA

            def body(k, _, t=t):
                off = pl.multiple_of(k * CHA, 8)
                pltpu.sync_copy(ps_hbm.at[q, t, pl.ds(off, CHA)], sidx0)
                pltpu.sync_copy(pd_hbm.at[q, t, pl.ds(off, CHA)], dbuf0)
                pltpu.async_copy(g_hbm.at[sidx0], rows0, gsem0).wait()
                pltpu.sync_copy(rows0, acc.at[dbuf0], add=True)
                return ()

            lax.fori_loop(0, nch, body, (), unroll=False)

        plsc.subcore_barrier()
        @pl.when(s < NS - 1)
        def _():
            pltpu.sync_copy(acc.at[pl.ds(r0, ACC_STRIPE), :],
                            out_hbm.at[pl.ds(c * QN + r0, ACC_STRIPE), :])

        @pl.when(s == NS - 1)
        def _():
            pltpu.sync_copy(acc.at[pl.ds(r0, last), :],
                            out_hbm.at[pl.ds(c * QN + r0, last), :])

    return functools.partial(
        pl.kernel,
        out_type=jax.ShapeDtypeStruct((HALF, DD), jnp.float32),
        mesh=_mesh,
        scratch_types=[
            pltpu.VMEM((CHA,), jnp.int32),
            pltpu.VMEM((CHA,), jnp.int32),
            pltpu.VMEM((LANES,), jnp.int32),
            pltpu.VMEM((CHA, DD), jnp.float32),
            pltpu.VMEM((98, DD), jnp.float32),
            pltpu.VMEM_SHARED((ACC_ROWS, DD), jnp.float32),
            pltpu.SemaphoreType.DMA,
        ],
        compiler_params=_sc_params,
    )(_agg_body)


_agg_p0 = _make_agg(0)
_agg_p1 = _make_agg(1)


def _agg(g, ps, pd, pc, zrows):
    lo = _agg_p0(g, ps, pd, pc, zrows)
    hi = _agg_p1(g, ps, pd, pc, zrows)
    return jnp.concatenate([lo, hi], axis=0)


BLK = 1000
GRID = NN // BLK


def _b1_body(x_ref, w_ref, deg_ref, g_ref, dinv_ref):
    deg = deg_ref[0, :, 0:1] + deg_ref[1, :, 0:1] + 1.0
    dinv = lax.rsqrt(deg)
    dinv_ref[...] = dinv
    g_ref[...] = jnp.dot(x_ref[...], w_ref[...],
                         preferred_element_type=jnp.float32) * dinv


def _b1(x, W0, deg2):
    return pl.pallas_call(
        _b1_body,
        grid=(GRID,),
        in_specs=[
            pl.BlockSpec((BLK, DD), lambda i: (i, 0)),
            pl.BlockSpec((DD, DD), lambda i: (0, 0)),
            pl.BlockSpec((NC, BLK, 16), lambda i: (0, i, 0)),
        ],
        out_specs=[
            pl.BlockSpec((BLK, DD), lambda i: (i, 0)),
            pl.BlockSpec((BLK, 1), lambda i: (i, 0)),
        ],
        out_shape=[
            jax.ShapeDtypeStruct((NN, DD), jnp.float32),
            jax.ShapeDtypeStruct((NN, 1), jnp.float32),
        ],
    )(x, W0, deg2)


def _b2_body(raw_ref, g_ref, dinv_ref, w_ref, b_ref, out_ref):
    dinv = dinv_ref[...]
    h1 = (raw_ref[...] + g_ref[...]) * dinv + b_ref[...]
    out_ref[...] = jnp.dot(h1, w_ref[...],
                           preferred_element_type=jnp.float32) * dinv


def _b2(raw1, g1, dinv, W1, b0):
    return pl.pallas_call(
        _b2_body,
        grid=(GRID,),
        in_specs=[
            pl.BlockSpec((BLK, DD), lambda i: (i, 0)),
            pl.BlockSpec((BLK, DD), lambda i: (i, 0)),
            pl.BlockSpec((BLK, 1), lambda i: (i, 0)),
            pl.BlockSpec((DD, DD), lambda i: (0, 0)),
            pl.BlockSpec((1, DD), lambda i: (0, 0)),
        ],
        out_specs=pl.BlockSpec((BLK, DD), lambda i: (i, 0)),
        out_shape=jax.ShapeDtypeStruct((NN, DD), jnp.float32),
    )(raw1, g1, dinv, W1, b0)


def _d1_body(raw_ref, g_ref, dinv_ref, b_ref, h_ref, stat_ref, acc_ref):
    i = pl.program_id(0)
    h = (raw_ref[...] + g_ref[...]) * dinv_ref[...] + b_ref[...]
    h_ref[...] = h
    ps = jnp.sum(h, 0, keepdims=True)
    pq = jnp.sum(h * h, 0, keepdims=True)
    blk = jnp.concatenate([ps, pq], 0)

    @pl.when(i == 0)
    def _():
        acc_ref[...] = blk

    @pl.when(i > 0)
    def _():
        acc_ref[...] += blk

    @pl.when(i == pl.num_programs(0) - 1)
    def _():
        stat_ref[...] = acc_ref[...]


def _d1(raw2, g2, dinv, b1):
    return pl.pallas_call(
        _d1_body,
        grid=(GRID,),
        in_specs=[
            pl.BlockSpec((BLK, DD), lambda i: (i, 0)),
            pl.BlockSpec((BLK, DD), lambda i: (i, 0)),
            pl.BlockSpec((BLK, 1), lambda i: (i, 0)),
            pl.BlockSpec((1, DD), lambda i: (0, 0)),
        ],
        out_specs=[
            pl.BlockSpec((BLK, DD), lambda i: (i, 0)),
            pl.BlockSpec((2, DD), lambda i: (0, 0)),
        ],
        out_shape=[
            jax.ShapeDtypeStruct((NN, DD), jnp.float32),
            jax.ShapeDtypeStruct((2, DD), jnp.float32),
        ],
        scratch_shapes=[pltpu.VMEM((2, DD), jnp.float32)],
    )(raw2, g2, dinv, b1)


def _d2_body(h_ref, stat_ref, gamma_ref, beta_ref, out_ref):
    mean = stat_ref[0:1] * (1.0 / NN)
    var = stat_ref[1:2] * (1.0 / NN) - mean * mean
    rstd = lax.rsqrt(var + 1e-5)
    out_ref[...] = (h_ref[...] - mean) * rstd * gamma_ref[...] + beta_ref[...]


def _d2(h2, stat, gamma, beta):
    return pl.pallas_call(
        _d2_body,
        grid=(GRID,),
        in_specs=[
            pl.BlockSpec((BLK, DD), lambda i: (i, 0)),
            pl.BlockSpec((2, DD), lambda i: (0, 0)),
            pl.BlockSpec((1, DD), lambda i: (0, 0)),
            pl.BlockSpec((1, DD), lambda i: (0, 0)),
        ],
        out_specs=pl.BlockSpec((BLK, DD), lambda i: (i, 0)),
        out_shape=jax.ShapeDtypeStruct((NN, DD), jnp.float32),
    )(h2, stat, gamma, beta)


def kernel(x, edge_index, W0, b0, W1, b1, gamma, beta):
    ei = edge_index.astype(jnp.int32)
    src, dst = ei[0], ei[1]
    ones = jnp.ones((CH, 16), jnp.float32)
    z1 = jnp.zeros((3128, 16), jnp.float32)
    zrows = jnp.zeros((98, DD), jnp.float32)

    deg2 = _deg_kernel(dst, ones, z1)
    ps, pd, pc = _part_kernel(src, dst)
    g1, dinv = _b1(x, W0, deg2)
    raw1 = _agg(g1, ps, pd, pc, zrows)
    g2 = _b2(raw1, g1, dinv, W1, b0.reshape(1, DD))
    raw2 = _agg(g2, ps, pd, pc, zrows)
    h2, stat = _d1(raw2, g2, dinv, b1.reshape(1, DD))
    out = _d2(h2, stat, gamma.reshape(1, DD), beta.reshape(1, DD))
    return (out[:NUSERS], out[NUSERS:])


# speedup vs baseline: 1.8014x; 1.8014x over previous
import functools

import jax
import jax.numpy as jnp
from jax import lax
from jax.experimental import pallas as pl
from jax.experimental.pallas import tpu as pltpu
from jax.experimental.pallas import tpu_sc as plsc

NUSERS = 20000
NN = 50000
DD = 64
EE = 800000
NC, NS, LANES = 2, 16, 16
NW = NC * NS
HALF = NN // 2
QN = NN // 4
TRASH = QN
ACC_STRIPE = 784
ACC_ROWS = NS * ACC_STRIPE
DEG_WORDS = 16 * 3128

CH = 128
CHA = 512
QCAP = 26624
FCH = 2048
PCH = 512
PER_W = EE // NW
NPF = PER_W // PCH
PTAIL = PER_W - NPF * PCH

_mesh = plsc.VectorSubcoreMesh(
    core_axis_name="c", subcore_axis_name="s", num_cores=NC, num_subcores=NS)
_sc_params = pltpu.CompilerParams(use_tc_tiling_on_sc=False,
                                  needs_layout_passes=False)


def _deg_body(dst_hbm, ones_hbm, z_hbm, deg_out, didx, didx_t, ones_v, zv,
              acc):
    c = lax.axis_index("c")
    s = lax.axis_index("s")
    pltpu.sync_copy(z_hbm, zv)
    pltpu.sync_copy(zv, acc.at[pl.ds(pl.multiple_of(s * 3128, 8), 3128), :])
    pltpu.sync_copy(ones_hbm, ones_v)
    plsc.subcore_barrier()

    base = (c * NS + s) * PER_W
    nfull = PER_W // CH
    tail = PER_W - nfull * CH

    def body(i, _):
        off = pl.multiple_of(base + i * CH, 8)
        pltpu.sync_copy(dst_hbm.at[pl.ds(off, CH)], didx)
        pltpu.sync_copy(ones_v, acc.at[didx, :], add=True)
        return ()

    lax.fori_loop(0, nfull, body, (), unroll=False)
    off = pl.multiple_of(base + nfull * CH, 8)
    pltpu.sync_copy(dst_hbm.at[pl.ds(off, tail)], didx_t)
    pltpu.sync_copy(ones_v.at[pl.ds(0, tail), :], acc.at[didx_t, :], add=True)

    plsc.subcore_barrier()
    nw = 3128
    w0 = pl.multiple_of(s * nw, 8)
    last = NN - 15 * nw

    @pl.when(s < NS - 1)
    def _():
        pltpu.sync_copy(acc.at[pl.ds(w0, nw), :],
                        deg_out.at[c, pl.ds(w0, nw), :])

    @pl.when(s == NS - 1)
    def _():
        pltpu.sync_copy(acc.at[pl.ds(w0, last), :],
                        deg_out.at[c, pl.ds(w0, last), :])


_deg_kernel = functools.partial(
    pl.kernel,
    out_type=jax.ShapeDtypeStruct((NC, NN, 16), jnp.float32),
    mesh=_mesh,
    scratch_types=[
        pltpu.VMEM((CH,), jnp.int32),
        pltpu.VMEM((40,), jnp.int32),
        pltpu.VMEM((CH, 16), jnp.float32),
        pltpu.VMEM((3128, 16), jnp.float32),
        pltpu.VMEM_SHARED((DEG_WORDS, 16), jnp.float32),
    ],
    compiler_params=_sc_params,
)(_deg_body)


def _part_body(src_hbm, dst_hbm, ps_out, pd_out, cnt_out,
               sbuf, dbuf, osa, oda, osb, odb, cbuf):
    c = lax.axis_index("c")
    s = lax.axis_index("s")
    t = c * NS + s
    tbase = t * PER_W

    lane = lax.iota(jnp.int32, 16)
    zero_v = jnp.zeros((LANES,), jnp.int32)
    trash_v = jnp.full((LANES,), TRASH, jnp.int32)

    for p in range(2):
        qa = 2 * p
        lo_a, lo_b = qa * QN, (qa + 1) * QN

        def scan_vregs(nv, last_lanes, ca, cb, qa=qa, lo_a=lo_a, lo_b=lo_b):
            for j in range(nv):
                sv = sbuf[pl.ds(j * LANES, LANES)]
                dv = dbuf[pl.ds(j * LANES, LANES)]
                in_a = (dv >= lo_a) & (dv < lo_b)
                in_b = (dv >= lo_b) & (dv < lo_b + QN)
                if j == nv - 1 and last_lanes < LANES:
                    valid = lane < last_lanes
                    in_a = in_a & valid
                    in_b = in_b & valid
                plsc.store_compressed(osa.at[pl.ds(ca, LANES)], sv, mask=in_a)
                plsc.store_compressed(oda.at[pl.ds(ca, LANES)], dv - lo_a,
                                      mask=in_a)
                plsc.store_compressed(osb.at[pl.ds(cb, LANES)], sv, mask=in_b)
                plsc.store_compressed(odb.at[pl.ds(cb, LANES)], dv - lo_b,
                                      mask=in_b)
                ca = ca + jnp.sum(in_a.astype(jnp.int32))
                cb = cb + jnp.sum(in_b.astype(jnp.int32))
            return ca, cb

        def chunk(i, carry, scan_vregs=scan_vregs):
            ca, cb = carry
            off = pl.multiple_of(tbase + i * PCH, 8)
            pltpu.sync_copy(src_hbm.at[pl.ds(off, PCH)], sbuf)
            pltpu.sync_copy(dst_hbm.at[pl.ds(off, PCH)], dbuf)
            return scan_vregs(PCH // LANES, LANES, ca, cb)

        z = jnp.int32(0)
        ca, cb = lax.fori_loop(0, NPF, chunk, (z, z), unroll=False)
        off = pl.multiple_of(tbase + NPF * PCH, 8)
        pltpu.sync_copy(src_hbm.at[pl.ds(off, PTAIL)],
                        sbuf.at[pl.ds(0, PTAIL)])
        pltpu.sync_copy(dst_hbm.at[pl.ds(off, PTAIL)],
                        dbuf.at[pl.ds(0, PTAIL)])
        ca, cb = scan_vregs(PTAIL // LANES + 1, PTAIL % LANES, ca, cb)

        for k in range(CHA // LANES):
            osa[pl.ds(ca + k * LANES, LANES)] = zero_v
            oda[pl.ds(ca + k * LANES, LANES)] = trash_v
            osb[pl.ds(cb + k * LANES, LANES)] = zero_v
            odb[pl.ds(cb + k * LANES, LANES)] = trash_v
        rca = (ca + CHA - 1) // CHA * CHA
        rcb = (cb + CHA - 1) // CHA * CHA

        cbuf[0, :] = jnp.full((LANES,), rca, jnp.int32)
        cbuf[1, :] = jnp.full((LANES,), rcb, jnp.int32)
        pltpu.sync_copy(cbuf.at[0], cnt_out.at[qa, t])
        pltpu.sync_copy(cbuf.at[1], cnt_out.at[qa + 1, t])

        def flush_a(k, _, qa=qa):
            o = pl.multiple_of(k * FCH, 8)
            pltpu.sync_copy(osa.at[pl.ds(o, FCH)],
                            ps_out.at[qa, t, pl.ds(o, FCH)])
            pltpu.sync_copy(oda.at[pl.ds(o, FCH)],
                            pd_out.at[qa, t, pl.ds(o, FCH)])
            return ()

        def flush_b(k, _, qa=qa):
            o = pl.multiple_of(k * FCH, 8)
            pltpu.sync_copy(osb.at[pl.ds(o, FCH)],
                            ps_out.at[qa + 1, t, pl.ds(o, FCH)])
            pltpu.sync_copy(odb.at[pl.ds(o, FCH)],
                            pd_out.at[qa + 1, t, pl.ds(o, FCH)])
            return ()

        lax.fori_loop(0, (rca + FCH - 1) // FCH, flush_a, (), unroll=False)
        lax.fori_loop(0, (rcb + FCH - 1) // FCH, flush_b, (), unroll=False)


_part_kernel = functools.partial(
    pl.kernel,
    out_type=[jax.ShapeDtypeStruct((4, NW, QCAP), jnp.int32),
              jax.ShapeDtypeStruct((4, NW, QCAP), jnp.int32),
              jax.ShapeDtypeStruct((4, NW, 16), jnp.int32)],
    mesh=_mesh,
    scratch_types=[
        pltpu.VMEM((PCH,), jnp.int32),
        pltpu.VMEM((PCH,), jnp.int32),
        pltpu.VMEM((QCAP,), jnp.int32),
        pltpu.VMEM((QCAP,), jnp.int32),
        pltpu.VMEM((QCAP,), jnp.int32),
        pltpu.VMEM((QCAP,), jnp.int32),
        pltpu.VMEM((2, LANES), jnp.int32),
    ],
    compiler_params=_sc_params,
)(_part_body)


def _make_agg(p):
    def _agg_body(g_hbm, ps_hbm, pd_hbm, pc_hbm, z_hbm, out_hbm,
                  sidx0, dbuf0, cbuf, rows0, zv, acc, gsem0):
        c = lax.axis_index("c")
        s = lax.axis_index("s")
        q = 2 * p + c

        r0 = s * ACC_STRIPE
        last = QN - 15 * ACC_STRIPE

        pltpu.sync_copy(z_hbm, zv)
        for k in range(8):
            pltpu.sync_copy(zv, acc.at[pl.ds(r0 + k * 98, 98), :])
        plsc.subcore_barrier()

        for r in range(2):
            t = s * 2 + r
            pltpu.sync_copy(pc_hbm.at[q, t], cbuf)
            rc = lax.reduce_max(cbuf[...], axes=(0,))
            nch = rc // CHA

            def body(k, _, t=t):
                off = pl.multiple_of(k * CHA, 8)
                pltpu.sync_copy(ps_hbm.at[q, t, pl.ds(off, CHA)], sidx0)
                pltpu.sync_copy(pd_hbm.at[q, t, pl.ds(off, CHA)], dbuf0)
                pltpu.async_copy(g_hbm.at[sidx0], rows0, gsem0).wait()
                pltpu.sync_copy(rows0, acc.at[dbuf0], add=True)
                return ()

            lax.fori_loop(0, nch, body, (), unroll=False)

        plsc.subcore_barrier()
        @pl.when(s < NS - 1)
        def _():
            pltpu.sync_copy(acc.at[pl.ds(r0, ACC_STRIPE), :],
                            out_hbm.at[pl.ds(c * QN + r0, ACC_STRIPE), :])

        @pl.when(s == NS - 1)
        def _():
            pltpu.sync_copy(acc.at[pl.ds(r0, last), :],
                            out_hbm.at[pl.ds(c * QN + r0, last), :])

    return functools.partial(
        pl.kernel,
        out_type=jax.ShapeDtypeStruct((HALF, DD), jnp.float32),
        mesh=_mesh,
        scratch_types=[
            pltpu.VMEM((CHA,), jnp.int32),
            pltpu.VMEM((CHA,), jnp.int32),
            pltpu.VMEM((LANES,), jnp.int32),
            pltpu.VMEM((CHA, DD), jnp.float32),
            pltpu.VMEM((98, DD), jnp.float32),
            pltpu.VMEM_SHARED((ACC_ROWS, DD), jnp.float32),
            pltpu.SemaphoreType.DMA,
        ],
        compiler_params=_sc_params,
    )(_agg_body)


_agg_p0 = _make_agg(0)
_agg_p1 = _make_agg(1)


def _agg(g, ps, pd, pc, zrows):
    lo = _agg_p0(g, ps, pd, pc, zrows)
    hi = _agg_p1(g, ps, pd, pc, zrows)
    return jnp.concatenate([lo, hi], axis=0)


BLK = 1000
GRID = NN // BLK


def _b1_body(x_ref, w_ref, deg_ref, g_ref, dinv_ref):
    deg = deg_ref[0, :, 0:1] + deg_ref[1, :, 0:1] + 1.0
    dinv = lax.rsqrt(deg)
    dinv_ref[...] = dinv
    g_ref[...] = jnp.dot(x_ref[...], w_ref[...],
                         preferred_element_type=jnp.float32) * dinv


def _b1(x, W0, deg2):
    return pl.pallas_call(
        _b1_body,
        grid=(GRID,),
        in_specs=[
            pl.BlockSpec((BLK, DD), lambda i: (i, 0)),
            pl.BlockSpec((DD, DD), lambda i: (0, 0)),
            pl.BlockSpec((NC, BLK, 16), lambda i: (0, i, 0)),
        ],
        out_specs=[
            pl.BlockSpec((BLK, DD), lambda i: (i, 0)),
            pl.BlockSpec((BLK, 1), lambda i: (i, 0)),
        ],
        out_shape=[
            jax.ShapeDtypeStruct((NN, DD), jnp.float32),
            jax.ShapeDtypeStruct((NN, 1), jnp.float32),
        ],
    )(x, W0, deg2)


def _b2_body(raw_ref, g_ref, dinv_ref, w_ref, b_ref, out_ref):
    dinv = dinv_ref[...]
    h1 = (raw_ref[...] + g_ref[...]) * dinv + b_ref[...]
    out_ref[...] = jnp.dot(h1, w_ref[...],
                           preferred_element_type=jnp.float32) * dinv


def _b2(raw1, g1, dinv, W1, b0):
    return pl.pallas_call(
        _b2_body,
        grid=(GRID,),
        in_specs=[
            pl.BlockSpec((BLK, DD), lambda i: (i, 0)),
            pl.BlockSpec((BLK, DD), lambda i: (i, 0)),
            pl.BlockSpec((BLK, 1), lambda i: (i, 0)),
            pl.BlockSpec((DD, DD), lambda i: (0, 0)),
            pl.BlockSpec((1, DD), lambda i: (0, 0)),
        ],
        out_specs=pl.BlockSpec((BLK, DD), lambda i: (i, 0)),
        out_shape=jax.ShapeDtypeStruct((NN, DD), jnp.float32),
    )(raw1, g1, dinv, W1, b0)


def _d1_body(raw_ref, g_ref, dinv_ref, b_ref, h_ref, stat_ref, acc_ref):
    i = pl.program_id(0)
    h = (raw_ref[...] + g_ref[...]) * dinv_ref[...] + b_ref[...]
    h_ref[...] = h
    ps = jnp.sum(h, 0, keepdims=True)
    pq = jnp.sum(h * h, 0, keepdims=True)
    blk = jnp.concatenate([ps, pq], 0)

    @pl.when(i == 0)
    def _():
        acc_ref[...] = blk

    @pl.when(i > 0)
    def _():
        acc_ref[...] += blk

    @pl.when(i == pl.num_programs(0) - 1)
    def _():
        stat_ref[...] = acc_ref[...]


def _d1(raw2, g2, dinv, b1):
    return pl.pallas_call(
        _d1_body,
        grid=(GRID,),
        in_specs=[
            pl.BlockSpec((BLK, DD), lambda i: (i, 0)),
            pl.BlockSpec((BLK, DD), lambda i: (i, 0)),
            pl.BlockSpec((BLK, 1), lambda i: (i, 0)),
            pl.BlockSpec((1, DD), lambda i: (0, 0)),
        ],
        out_specs=[
            pl.BlockSpec((BLK, DD), lambda i: (i, 0)),
            pl.BlockSpec((2, DD), lambda i: (0, 0)),
        ],
        out_shape=[
            jax.ShapeDtypeStruct((NN, DD), jnp.float32),
            jax.ShapeDtypeStruct((2, DD), jnp.float32),
        ],
        scratch_shapes=[pltpu.VMEM((2, DD), jnp.float32)],
    )(raw2, g2, dinv, b1)


def _d2_body(h_ref, stat_ref, gamma_ref, beta_ref, out_ref):
    mean = stat_ref[0:1] * (1.0 / NN)
    var = stat_ref[1:2] * (1.0 / NN) - mean * mean
    rstd = lax.rsqrt(var + 1e-5)
    out_ref[...] = (h_ref[...] - mean) * rstd * gamma_ref[...] + beta_ref[...]


def _d2(h2, stat, gamma, beta):
    return pl.pallas_call(
        _d2_body,
        grid=(GRID,),
        in_specs=[
            pl.BlockSpec((BLK, DD), lambda i: (i, 0)),
            pl.BlockSpec((2, DD), lambda i: (0, 0)),
            pl.BlockSpec((1, DD), lambda i: (0, 0)),
            pl.BlockSpec((1, DD), lambda i: (0, 0)),
        ],
        out_specs=pl.BlockSpec((BLK, DD), lambda i: (i, 0)),
        out_shape=jax.ShapeDtypeStruct((NN, DD), jnp.float32),
    )(h2, stat, gamma, beta)


def kernel(x, edge_index, W0, b0, W1, b1, gamma, beta):
    ei = edge_index.astype(jnp.int32)
    src, dst = ei[0], ei[1]
    ones = jnp.ones((CH, 16), jnp.float32)
    z1 = jnp.zeros((3128, 16), jnp.float32)
    zrows = jnp.zeros((98, DD), jnp.float32)

    deg2 = _deg_kernel(dst, ones, z1)
    ps, pd, pc = _part_kernel(src, dst)
    g1, dinv = _b1(x, W0, deg2)
    raw1 = _agg(g1, ps, pd, pc, zrows)
    g2 = _b2(raw1, g1, dinv, W1, b0.reshape(1, DD))
    raw2 = _agg(g2, ps, pd, pc, zrows)
    h2, stat = _d1(raw2, g2, dinv, b1.reshape(1, DD))
    out = _d2(h2, stat, gamma.reshape(1, DD), beta.reshape(1, DD))
    return (out[:NUSERS], out[NUSERS:])


# speedup vs baseline: 2.7506x; 1.5269x over previous
import functools

import jax
import jax.numpy as jnp
from jax import lax
from jax.experimental import pallas as pl
from jax.experimental.pallas import tpu as pltpu
from jax.experimental.pallas import tpu_sc as plsc

NUSERS = 20000
NN = 50000
DD = 64
EE = 800000
NC, NS, LANES = 2, 16, 16
NW = NC * NS
HALF = NN // 2
QN = NN // 4
TRASH = QN
ACC_STRIPE = 784
ACC_ROWS = NS * ACC_STRIPE
DEG_WORDS = 16 * 3128

CH = 128
CHA = 256
QCAP = 26624
FCH = 2048
PCH = 512
PER_W = EE // NW
NPF = PER_W // PCH
PTAIL = PER_W - NPF * PCH

_mesh = plsc.VectorSubcoreMesh(
    core_axis_name="c", subcore_axis_name="s", num_cores=NC, num_subcores=NS)
_sc_params = pltpu.CompilerParams(use_tc_tiling_on_sc=False,
                                  needs_layout_passes=False)


def _deg_body(dst_hbm, ones_hbm, z_hbm, deg_out, didx, didx_t, ones_v, zv,
              acc):
    c = lax.axis_index("c")
    s = lax.axis_index("s")
    pltpu.sync_copy(z_hbm, zv)
    pltpu.sync_copy(zv, acc.at[pl.ds(pl.multiple_of(s * 3128, 8), 3128), :])
    pltpu.sync_copy(ones_hbm, ones_v)
    plsc.subcore_barrier()

    base = (c * NS + s) * PER_W
    nfull = PER_W // CH
    tail = PER_W - nfull * CH

    def body(i, _):
        off = pl.multiple_of(base + i * CH, 8)
        pltpu.sync_copy(dst_hbm.at[pl.ds(off, CH)], didx)
        pltpu.sync_copy(ones_v, acc.at[didx, :], add=True)
        return ()

    lax.fori_loop(0, nfull, body, (), unroll=False)
    off = pl.multiple_of(base + nfull * CH, 8)
    pltpu.sync_copy(dst_hbm.at[pl.ds(off, tail)], didx_t)
    pltpu.sync_copy(ones_v.at[pl.ds(0, tail), :], acc.at[didx_t, :], add=True)

    plsc.subcore_barrier()
    nw = 3128
    w0 = pl.multiple_of(s * nw, 8)
    last = NN - 15 * nw

    @pl.when(s < NS - 1)
    def _():
        pltpu.sync_copy(acc.at[pl.ds(w0, nw), :],
                        deg_out.at[c, pl.ds(w0, nw), :])

    @pl.when(s == NS - 1)
    def _():
        pltpu.sync_copy(acc.at[pl.ds(w0, last), :],
                        deg_out.at[c, pl.ds(w0, last), :])


_deg_kernel = functools.partial(
    pl.kernel,
    out_type=jax.ShapeDtypeStruct((NC, NN, 16), jnp.float32),
    mesh=_mesh,
    scratch_types=[
        pltpu.VMEM((CH,), jnp.int32),
        pltpu.VMEM((40,), jnp.int32),
        pltpu.VMEM((CH, 16), jnp.float32),
        pltpu.VMEM((3128, 16), jnp.float32),
        pltpu.VMEM_SHARED((DEG_WORDS, 16), jnp.float32),
    ],
    compiler_params=_sc_params,
)(_deg_body)


def _part_body(src_hbm, dst_hbm, ps_out, pd_out, cnt_out,
               sbuf, dbuf, osa, oda, osb, odb, cbuf):
    c = lax.axis_index("c")
    s = lax.axis_index("s")
    t = c * NS + s
    tbase = t * PER_W

    lane = lax.iota(jnp.int32, 16)
    zero_v = jnp.zeros((LANES,), jnp.int32)
    trash_v = jnp.full((LANES,), TRASH, jnp.int32)

    for p in range(2):
        qa = 2 * p
        lo_a, lo_b = qa * QN, (qa + 1) * QN

        def scan_vregs(nv, last_lanes, ca, cb, qa=qa, lo_a=lo_a, lo_b=lo_b):
            for j in range(nv):
                sv = sbuf[pl.ds(j * LANES, LANES)]
                dv = dbuf[pl.ds(j * LANES, LANES)]
                in_a = (dv >= lo_a) & (dv < lo_b)
                in_b = (dv >= lo_b) & (dv < lo_b + QN)
                if j == nv - 1 and last_lanes < LANES:
                    valid = lane < last_lanes
                    in_a = in_a & valid
                    in_b = in_b & valid
                plsc.store_compressed(osa.at[pl.ds(ca, LANES)], sv, mask=in_a)
                plsc.store_compressed(oda.at[pl.ds(ca, LANES)], dv - lo_a,
                                      mask=in_a)
                plsc.store_compressed(osb.at[pl.ds(cb, LANES)], sv, mask=in_b)
                plsc.store_compressed(odb.at[pl.ds(cb, LANES)], dv - lo_b,
                                      mask=in_b)
                ca = ca + jnp.sum(in_a.astype(jnp.int32))
                cb = cb + jnp.sum(in_b.astype(jnp.int32))
            return ca, cb

        def chunk(i, carry, scan_vregs=scan_vregs):
            ca, cb = carry
            off = pl.multiple_of(tbase + i * PCH, 8)
            pltpu.sync_copy(src_hbm.at[pl.ds(off, PCH)], sbuf)
            pltpu.sync_copy(dst_hbm.at[pl.ds(off, PCH)], dbuf)
            return scan_vregs(PCH // LANES, LANES, ca, cb)

        z = jnp.int32(0)
        ca, cb = lax.fori_loop(0, NPF, chunk, (z, z), unroll=False)
        off = pl.multiple_of(tbase + NPF * PCH, 8)
        pltpu.sync_copy(src_hbm.at[pl.ds(off, PTAIL)],
                        sbuf.at[pl.ds(0, PTAIL)])
        pltpu.sync_copy(dst_hbm.at[pl.ds(off, PTAIL)],
                        dbuf.at[pl.ds(0, PTAIL)])
        ca, cb = scan_vregs(PTAIL // LANES + 1, PTAIL % LANES, ca, cb)

        for k in range(CHA // LANES):
            osa[pl.ds(ca + k * LANES, LANES)] = zero_v
            oda[pl.ds(ca + k * LANES, LANES)] = trash_v
            osb[pl.ds(cb + k * LANES, LANES)] = zero_v
            odb[pl.ds(cb + k * LANES, LANES)] = trash_v
        rca = (ca + CHA - 1) // CHA * CHA
        rcb = (cb + CHA - 1) // CHA * CHA

        cbuf[0, :] = jnp.full((LANES,), rca, jnp.int32)
        cbuf[1, :] = jnp.full((LANES,), rcb, jnp.int32)
        pltpu.sync_copy(cbuf.at[0], cnt_out.at[qa, t])
        pltpu.sync_copy(cbuf.at[1], cnt_out.at[qa + 1, t])

        def flush_a(k, _, qa=qa):
            o = pl.multiple_of(k * FCH, 8)
            pltpu.sync_copy(osa.at[pl.ds(o, FCH)],
                            ps_out.at[qa, t, pl.ds(o, FCH)])
            pltpu.sync_copy(oda.at[pl.ds(o, FCH)],
                            pd_out.at[qa, t, pl.ds(o, FCH)])
            return ()

        def flush_b(k, _, qa=qa):
            o = pl.multiple_of(k * FCH, 8)
            pltpu.sync_copy(osb.at[pl.ds(o, FCH)],
                            ps_out.at[qa + 1, t, pl.ds(o, FCH)])
            pltpu.sync_copy(odb.at[pl.ds(o, FCH)],
                            pd_out.at[qa + 1, t, pl.ds(o, FCH)])
            return ()

        lax.fori_loop(0, (rca + FCH - 1) // FCH, flush_a, (), unroll=False)
        lax.fori_loop(0, (rcb + FCH - 1) // FCH, flush_b, (), unroll=False)


_part_kernel = functools.partial(
    pl.kernel,
    out_type=[jax.ShapeDtypeStruct((4, NW, QCAP), jnp.int32),
              jax.ShapeDtypeStruct((4, NW, QCAP), jnp.int32),
              jax.ShapeDtypeStruct((4, NW, 16), jnp.int32)],
    mesh=_mesh,
    scratch_types=[
        pltpu.VMEM((PCH,), jnp.int32),
        pltpu.VMEM((PCH,), jnp.int32),
        pltpu.VMEM((QCAP,), jnp.int32),
        pltpu.VMEM((QCAP,), jnp.int32),
        pltpu.VMEM((QCAP,), jnp.int32),
        pltpu.VMEM((QCAP,), jnp.int32),
        pltpu.VMEM((2, LANES), jnp.int32),
    ],
    compiler_params=_sc_params,
)(_part_body)


def _make_agg(p):
    def _agg_body(g_hbm, ps_hbm, pd_hbm, pc_hbm, z_hbm, out_hbm,
                  sidx0, dbuf0, cbuf, rows0, zv, acc, gsem0):
        c = lax.axis_index("c")
        s = lax.axis_index("s")
        q = 2 * p + c

        r0 = s * ACC_STRIPE
        last = QN - 15 * ACC_STRIPE

        pltpu.sync_copy(z_hbm, zv)
        for k in range(8):
            pltpu.sync_copy(zv, acc.at[pl.ds(r0 + k * 98, 98), :])
        plsc.subcore_barrier()

        for r in range(2):
            t = s * 2 + r
            pltpu.sync_copy(pc_hbm.at[q, t], cbuf)
            rc = lax.reduce_max(cbuf[...], axes=(0,))
            nch = rc // CHA

            def body(k, _, t=t):
                off = pl.multiple_of(k * CHA, 8)
                pltpu.sync_copy(ps_hbm.at[q, t, pl.ds(off, CHA)], sidx0)
                pltpu.sync_copy(pd_hbm.at[q, t, pl.ds(off, CHA)], dbuf0)
                pltpu.async_copy(g_hbm.at[sidx0], rows0, gsem0).wait()
                pltpu.sync_copy(rows0, acc.at[dbuf0], add=True)
                return ()

            lax.fori_loop(0, nch, body, (), unroll=False)

        plsc.subcore_barrier()
        @pl.when(s < NS - 1)
        def _():
            pltpu.sync_copy(acc.at[pl.ds(r0, ACC_STRIPE), :],
                            out_hbm.at[pl.ds(c * QN + r0, ACC_STRIPE), :])

        @pl.when(s == NS - 1)
        def _():
            pltpu.sync_copy(acc.at[pl.ds(r0, last), :],
                            out_hbm.at[pl.ds(c * QN + r0, last), :])

    return functools.partial(
        pl.kernel,
        out_type=jax.ShapeDtypeStruct((HALF, DD), jnp.float32),
        mesh=_mesh,
        scratch_types=[
            pltpu.VMEM((CHA,), jnp.int32),
            pltpu.VMEM((CHA,), jnp.int32),
            pltpu.VMEM((LANES,), jnp.int32),
            pltpu.VMEM((CHA, DD), jnp.float32),
            pltpu.VMEM((98, DD), jnp.float32),
            pltpu.VMEM_SHARED((ACC_ROWS, DD), jnp.float32),
            pltpu.SemaphoreType.DMA,
        ],
        compiler_params=_sc_params,
    )(_agg_body)


_agg_p0 = _make_agg(0)
_agg_p1 = _make_agg(1)


def _agg(g, ps, pd, pc, zrows):
    lo = _agg_p0(g, ps, pd, pc, zrows)
    hi = _agg_p1(g, ps, pd, pc, zrows)
    return jnp.concatenate([lo, hi], axis=0)


BLK = 1000
GRID = NN // BLK


def _b1_body(x_ref, w_ref, deg_ref, g_ref, dinv_ref):
    deg = deg_ref[0, :, 0:1] + deg_ref[1, :, 0:1] + 1.0
    dinv = lax.rsqrt(deg)
    dinv_ref[...] = dinv
    g_ref[...] = jnp.dot(x_ref[...], w_ref[...],
                         preferred_element_type=jnp.float32) * dinv


def _b1(x, W0, deg2):
    return pl.pallas_call(
        _b1_body,
        grid=(GRID,),
        in_specs=[
            pl.BlockSpec((BLK, DD), lambda i: (i, 0)),
            pl.BlockSpec((DD, DD), lambda i: (0, 0)),
            pl.BlockSpec((NC, BLK, 16), lambda i: (0, i, 0)),
        ],
        out_specs=[
            pl.BlockSpec((BLK, DD), lambda i: (i, 0)),
            pl.BlockSpec((BLK, 1), lambda i: (i, 0)),
        ],
        out_shape=[
            jax.ShapeDtypeStruct((NN, DD), jnp.float32),
            jax.ShapeDtypeStruct((NN, 1), jnp.float32),
        ],
    )(x, W0, deg2)


def _b2_body(raw_ref, g_ref, dinv_ref, w_ref, b_ref, out_ref):
    dinv = dinv_ref[...]
    h1 = (raw_ref[...] + g_ref[...]) * dinv + b_ref[...]
    out_ref[...] = jnp.dot(h1, w_ref[...],
                           preferred_element_type=jnp.float32) * dinv


def _b2(raw1, g1, dinv, W1, b0):
    return pl.pallas_call(
        _b2_body,
        grid=(GRID,),
        in_specs=[
            pl.BlockSpec((BLK, DD), lambda i: (i, 0)),
            pl.BlockSpec((BLK, DD), lambda i: (i, 0)),
            pl.BlockSpec((BLK, 1), lambda i: (i, 0)),
            pl.BlockSpec((DD, DD), lambda i: (0, 0)),
            pl.BlockSpec((1, DD), lambda i: (0, 0)),
        ],
        out_specs=pl.BlockSpec((BLK, DD), lambda i: (i, 0)),
        out_shape=jax.ShapeDtypeStruct((NN, DD), jnp.float32),
    )(raw1, g1, dinv, W1, b0)


def _d1_body(raw_ref, g_ref, dinv_ref, b_ref, h_ref, stat_ref, acc_ref):
    i = pl.program_id(0)
    h = (raw_ref[...] + g_ref[...]) * dinv_ref[...] + b_ref[...]
    h_ref[...] = h
    ps = jnp.sum(h, 0, keepdims=True)
    pq = jnp.sum(h * h, 0, keepdims=True)
    blk = jnp.concatenate([ps, pq], 0)

    @pl.when(i == 0)
    def _():
        acc_ref[...] = blk

    @pl.when(i > 0)
    def _():
        acc_ref[...] += blk

    @pl.when(i == pl.num_programs(0) - 1)
    def _():
        stat_ref[...] = acc_ref[...]


def _d1(raw2, g2, dinv, b1):
    return pl.pallas_call(
        _d1_body,
        grid=(GRID,),
        in_specs=[
            pl.BlockSpec((BLK, DD), lambda i: (i, 0)),
            pl.BlockSpec((BLK, DD), lambda i: (i, 0)),
            pl.BlockSpec((BLK, 1), lambda i: (i, 0)),
            pl.BlockSpec((1, DD), lambda i: (0, 0)),
        ],
        out_specs=[
            pl.BlockSpec((BLK, DD), lambda i: (i, 0)),
            pl.BlockSpec((2, DD), lambda i: (0, 0)),
        ],
        out_shape=[
            jax.ShapeDtypeStruct((NN, DD), jnp.float32),
            jax.ShapeDtypeStruct((2, DD), jnp.float32),
        ],
        scratch_shapes=[pltpu.VMEM((2, DD), jnp.float32)],
    )(raw2, g2, dinv, b1)


def _d2_body(h_ref, stat_ref, gamma_ref, beta_ref, out_ref):
    mean = stat_ref[0:1] * (1.0 / NN)
    var = stat_ref[1:2] * (1.0 / NN) - mean * mean
    rstd = lax.rsqrt(var + 1e-5)
    out_ref[...] = (h_ref[...] - mean) * rstd * gamma_ref[...] + beta_ref[...]


def _d2(h2, stat, gamma, beta):
    return pl.pallas_call(
        _d2_body,
        grid=(GRID,),
        in_specs=[
            pl.BlockSpec((BLK, DD), lambda i: (i, 0)),
            pl.BlockSpec((2, DD), lambda i: (0, 0)),
            pl.BlockSpec((1, DD), lambda i: (0, 0)),
            pl.BlockSpec((1, DD), lambda i: (0, 0)),
        ],
        out_specs=pl.BlockSpec((BLK, DD), lambda i: (i, 0)),
        out_shape=jax.ShapeDtypeStruct((NN, DD), jnp.float32),
    )(h2, stat, gamma, beta)


def kernel(x, edge_index, W0, b0, W1, b1, gamma, beta):
    ei = edge_index.astype(jnp.int32)
    src, dst = ei[0], ei[1]
    ones = jnp.ones((CH, 16), jnp.float32)
    z1 = jnp.zeros((3128, 16), jnp.float32)
    zrows = jnp.zeros((98, DD), jnp.float32)

    deg2 = _deg_kernel(dst, ones, z1)
    ps, pd, pc = _part_kernel(src, dst)
    g1, dinv = _b1(x, W0, deg2)
    raw1 = _agg(g1, ps, pd, pc, zrows)
    g2 = _b2(raw1, g1, dinv, W1, b0.reshape(1, DD))
    raw2 = _agg(g2, ps, pd, pc, zrows)
    h2, stat = _d1(raw2, g2, dinv, b1.reshape(1, DD))
    out = _d2(h2, stat, gamma.reshape(1, DD), beta.reshape(1, DD))
    return (out[:NUSERS], out[NUSERS:])


# speedup vs baseline: 2.8406x; 1.0327x over previous
import functools

import jax
import jax.numpy as jnp
from jax import lax
from jax.experimental import pallas as pl
from jax.experimental.pallas import tpu as pltpu
from jax.experimental.pallas import tpu_sc as plsc

NUSERS = 20000
NN = 50000
DD = 64
EE = 800000
NC, NS, LANES = 2, 16, 16
NW = NC * NS
HALF = NN // 2
QN = NN // 4
TRASH = QN
ACC_STRIPE = 784
ACC_ROWS = NS * ACC_STRIPE
DEG_WORDS = 16 * 3128

CH = 128
CHA = 128
QCAP = 26624
FCH = 2048
PCH = 512
PER_W = EE // NW
NPF = PER_W // PCH
PTAIL = PER_W - NPF * PCH

_mesh = plsc.VectorSubcoreMesh(
    core_axis_name="c", subcore_axis_name="s", num_cores=NC, num_subcores=NS)
_sc_params = pltpu.CompilerParams(use_tc_tiling_on_sc=False,
                                  needs_layout_passes=False)


def _deg_body(dst_hbm, ones_hbm, z_hbm, deg_out, didx, didx_t, ones_v, zv,
              acc):
    c = lax.axis_index("c")
    s = lax.axis_index("s")
    pltpu.sync_copy(z_hbm, zv)
    pltpu.sync_copy(zv, acc.at[pl.ds(pl.multiple_of(s * 3128, 8), 3128), :])
    pltpu.sync_copy(ones_hbm, ones_v)
    plsc.subcore_barrier()

    base = (c * NS + s) * PER_W
    nfull = PER_W // CH
    tail = PER_W - nfull * CH

    def body(i, _):
        off = pl.multiple_of(base + i * CH, 8)
        pltpu.sync_copy(dst_hbm.at[pl.ds(off, CH)], didx)
        pltpu.sync_copy(ones_v, acc.at[didx, :], add=True)
        return ()

    lax.fori_loop(0, nfull, body, (), unroll=False)
    off = pl.multiple_of(base + nfull * CH, 8)
    pltpu.sync_copy(dst_hbm.at[pl.ds(off, tail)], didx_t)
    pltpu.sync_copy(ones_v.at[pl.ds(0, tail), :], acc.at[didx_t, :], add=True)

    plsc.subcore_barrier()
    nw = 3128
    w0 = pl.multiple_of(s * nw, 8)
    last = NN - 15 * nw

    @pl.when(s < NS - 1)
    def _():
        pltpu.sync_copy(acc.at[pl.ds(w0, nw), :],
                        deg_out.at[c, pl.ds(w0, nw), :])

    @pl.when(s == NS - 1)
    def _():
        pltpu.sync_copy(acc.at[pl.ds(w0, last), :],
                        deg_out.at[c, pl.ds(w0, last), :])


_deg_kernel = functools.partial(
    pl.kernel,
    out_type=jax.ShapeDtypeStruct((NC, NN, 16), jnp.float32),
    mesh=_mesh,
    scratch_types=[
        pltpu.VMEM((CH,), jnp.int32),
        pltpu.VMEM((40,), jnp.int32),
        pltpu.VMEM((CH, 16), jnp.float32),
        pltpu.VMEM((3128, 16), jnp.float32),
        pltpu.VMEM_SHARED((DEG_WORDS, 16), jnp.float32),
    ],
    compiler_params=_sc_params,
)(_deg_body)


def _part_body(src_hbm, dst_hbm, ps_out, pd_out, cnt_out,
               sbuf, dbuf, osa, oda, osb, odb, cbuf):
    c = lax.axis_index("c")
    s = lax.axis_index("s")
    t = c * NS + s
    tbase = t * PER_W

    lane = lax.iota(jnp.int32, 16)
    zero_v = jnp.zeros((LANES,), jnp.int32)
    trash_v = jnp.full((LANES,), TRASH, jnp.int32)

    for p in range(2):
        qa = 2 * p
        lo_a, lo_b = qa * QN, (qa + 1) * QN

        def scan_vregs(nv, last_lanes, ca, cb, qa=qa, lo_a=lo_a, lo_b=lo_b):
            for j in range(nv):
                sv = sbuf[pl.ds(j * LANES, LANES)]
                dv = dbuf[pl.ds(j * LANES, LANES)]
                in_a = (dv >= lo_a) & (dv < lo_b)
                in_b = (dv >= lo_b) & (dv < lo_b + QN)
                if j == nv - 1 and last_lanes < LANES:
                    valid = lane < last_lanes
                    in_a = in_a & valid
                    in_b = in_b & valid
                plsc.store_compressed(osa.at[pl.ds(ca, LANES)], sv, mask=in_a)
                plsc.store_compressed(oda.at[pl.ds(ca, LANES)], dv - lo_a,
                                      mask=in_a)
                plsc.store_compressed(osb.at[pl.ds(cb, LANES)], sv, mask=in_b)
                plsc.store_compressed(odb.at[pl.ds(cb, LANES)], dv - lo_b,
                                      mask=in_b)
                ca = ca + jnp.sum(in_a.astype(jnp.int32))
                cb = cb + jnp.sum(in_b.astype(jnp.int32))
            return ca, cb

        def chunk(i, carry, scan_vregs=scan_vregs):
            ca, cb = carry
            off = pl.multiple_of(tbase + i * PCH, 8)
            pltpu.sync_copy(src_hbm.at[pl.ds(off, PCH)], sbuf)
            pltpu.sync_copy(dst_hbm.at[pl.ds(off, PCH)], dbuf)
            return scan_vregs(PCH // LANES, LANES, ca, cb)

        z = jnp.int32(0)
        ca, cb = lax.fori_loop(0, NPF, chunk, (z, z), unroll=False)
        off = pl.multiple_of(tbase + NPF * PCH, 8)
        pltpu.sync_copy(src_hbm.at[pl.ds(off, PTAIL)],
                        sbuf.at[pl.ds(0, PTAIL)])
        pltpu.sync_copy(dst_hbm.at[pl.ds(off, PTAIL)],
                        dbuf.at[pl.ds(0, PTAIL)])
        ca, cb = scan_vregs(PTAIL // LANES + 1, PTAIL % LANES, ca, cb)

        for k in range(CHA // LANES):
            osa[pl.ds(ca + k * LANES, LANES)] = zero_v
            oda[pl.ds(ca + k * LANES, LANES)] = trash_v
            osb[pl.ds(cb + k * LANES, LANES)] = zero_v
            odb[pl.ds(cb + k * LANES, LANES)] = trash_v
        rca = (ca + CHA - 1) // CHA * CHA
        rcb = (cb + CHA - 1) // CHA * CHA

        cbuf[0, :] = jnp.full((LANES,), rca, jnp.int32)
        cbuf[1, :] = jnp.full((LANES,), rcb, jnp.int32)
        pltpu.sync_copy(cbuf.at[0], cnt_out.at[qa, t])
        pltpu.sync_copy(cbuf.at[1], cnt_out.at[qa + 1, t])

        def flush_a(k, _, qa=qa):
            o = pl.multiple_of(k * FCH, 8)
            pltpu.sync_copy(osa.at[pl.ds(o, FCH)],
                            ps_out.at[qa, t, pl.ds(o, FCH)])
            pltpu.sync_copy(oda.at[pl.ds(o, FCH)],
                            pd_out.at[qa, t, pl.ds(o, FCH)])
            return ()

        def flush_b(k, _, qa=qa):
            o = pl.multiple_of(k * FCH, 8)
            pltpu.sync_copy(osb.at[pl.ds(o, FCH)],
                            ps_out.at[qa + 1, t, pl.ds(o, FCH)])
            pltpu.sync_copy(odb.at[pl.ds(o, FCH)],
                            pd_out.at[qa + 1, t, pl.ds(o, FCH)])
            return ()

        lax.fori_loop(0, (rca + FCH - 1) // FCH, flush_a, (), unroll=False)
        lax.fori_loop(0, (rcb + FCH - 1) // FCH, flush_b, (), unroll=False)


_part_kernel = functools.partial(
    pl.kernel,
    out_type=[jax.ShapeDtypeStruct((4, NW, QCAP), jnp.int32),
              jax.ShapeDtypeStruct((4, NW, QCAP), jnp.int32),
              jax.ShapeDtypeStruct((4, NW, 16), jnp.int32)],
    mesh=_mesh,
    scratch_types=[
        pltpu.VMEM((PCH,), jnp.int32),
        pltpu.VMEM((PCH,), jnp.int32),
        pltpu.VMEM((QCAP,), jnp.int32),
        pltpu.VMEM((QCAP,), jnp.int32),
        pltpu.VMEM((QCAP,), jnp.int32),
        pltpu.VMEM((QCAP,), jnp.int32),
        pltpu.VMEM((2, LANES), jnp.int32),
    ],
    compiler_params=_sc_params,
)(_part_body)


def _make_agg(p):
    def _agg_body(g_hbm, ps_hbm, pd_hbm, pc_hbm, z_hbm, out_hbm,
                  sidx0, dbuf0, cbuf, rows0, zv, acc, gsem0):
        c = lax.axis_index("c")
        s = lax.axis_index("s")
        q = 2 * p + c

        r0 = s * ACC_STRIPE
        last = QN - 15 * ACC_STRIPE

        pltpu.sync_copy(z_hbm, zv)
        for k in range(8):
            pltpu.sync_copy(zv, acc.at[pl.ds(r0 + k * 98, 98), :])
        plsc.subcore_barrier()

        for r in range(2):
            t = s * 2 + r
            pltpu.sync_copy(pc_hbm.at[q, t], cbuf)
            rc = lax.reduce_max(cbuf[...], axes=(0,))
            nch = rc // CHA

            def body(k, _, t=t):
                off = pl.multiple_of(k * CHA, 8)
                pltpu.sync_copy(ps_hbm.at[q, t, pl.ds(off, CHA)], sidx0)
                pltpu.sync_copy(pd_hbm.at[q, t, pl.ds(off, CHA)], dbuf0)
                pltpu.async_copy(g_hbm.at[sidx0], rows0, gsem0).wait()
                pltpu.sync_copy(rows0, acc.at[dbuf0], add=True)
                return ()

            lax.fori_loop(0, nch, body, (), unroll=False)

        plsc.subcore_barrier()
        @pl.when(s < NS - 1)
        def _():
            pltpu.sync_copy(acc.at[pl.ds(r0, ACC_STRIPE), :],
                            out_hbm.at[pl.ds(c * QN + r0, ACC_STRIPE), :])

        @pl.when(s == NS - 1)
        def _():
            pltpu.sync_copy(acc.at[pl.ds(r0, last), :],
                            out_hbm.at[pl.ds(c * QN + r0, last), :])

    return functools.partial(
        pl.kernel,
        out_type=jax.ShapeDtypeStruct((HALF, DD), jnp.float32),
        mesh=_mesh,
        scratch_types=[
            pltpu.VMEM((CHA,), jnp.int32),
            pltpu.VMEM((CHA,), jnp.int32),
            pltpu.VMEM((LANES,), jnp.int32),
            pltpu.VMEM((CHA, DD), jnp.float32),
            pltpu.VMEM((98, DD), jnp.float32),
            pltpu.VMEM_SHARED((ACC_ROWS, DD), jnp.float32),
            pltpu.SemaphoreType.DMA,
        ],
        compiler_params=_sc_params,
    )(_agg_body)


_agg_p0 = _make_agg(0)
_agg_p1 = _make_agg(1)


def _agg(g, ps, pd, pc, zrows):
    lo = _agg_p0(g, ps, pd, pc, zrows)
    hi = _agg_p1(g, ps, pd, pc, zrows)
    return jnp.concatenate([lo, hi], axis=0)


BLK = 1000
GRID = NN // BLK


def _b1_body(x_ref, w_ref, deg_ref, g_ref, dinv_ref):
    deg = deg_ref[0, :, 0:1] + deg_ref[1, :, 0:1] + 1.0
    dinv = lax.rsqrt(deg)
    dinv_ref[...] = dinv
    g_ref[...] = jnp.dot(x_ref[...], w_ref[...],
                         preferred_element_type=jnp.float32) * dinv


def _b1(x, W0, deg2):
    return pl.pallas_call(
        _b1_body,
        grid=(GRID,),
        in_specs=[
            pl.BlockSpec((BLK, DD), lambda i: (i, 0)),
            pl.BlockSpec((DD, DD), lambda i: (0, 0)),
            pl.BlockSpec((NC, BLK, 16), lambda i: (0, i, 0)),
        ],
        out_specs=[
            pl.BlockSpec((BLK, DD), lambda i: (i, 0)),
            pl.BlockSpec((BLK, 1), lambda i: (i, 0)),
        ],
        out_shape=[
            jax.ShapeDtypeStruct((NN, DD), jnp.float32),
            jax.ShapeDtypeStruct((NN, 1), jnp.float32),
        ],
    )(x, W0, deg2)


def _b2_body(raw_ref, g_ref, dinv_ref, w_ref, b_ref, out_ref):
    dinv = dinv_ref[...]
    h1 = (raw_ref[...] + g_ref[...]) * dinv + b_ref[...]
    out_ref[...] = jnp.dot(h1, w_ref[...],
                           preferred_element_type=jnp.float32) * dinv


def _b2(raw1, g1, dinv, W1, b0):
    return pl.pallas_call(
        _b2_body,
        grid=(GRID,),
        in_specs=[
            pl.BlockSpec((BLK, DD), lambda i: (i, 0)),
            pl.BlockSpec((BLK, DD), lambda i: (i, 0)),
            pl.BlockSpec((BLK, 1), lambda i: (i, 0)),
            pl.BlockSpec((DD, DD), lambda i: (0, 0)),
            pl.BlockSpec((1, DD), lambda i: (0, 0)),
        ],
        out_specs=pl.BlockSpec((BLK, DD), lambda i: (i, 0)),
        out_shape=jax.ShapeDtypeStruct((NN, DD), jnp.float32),
    )(raw1, g1, dinv, W1, b0)


def _d1_body(raw_ref, g_ref, dinv_ref, b_ref, h_ref, stat_ref, acc_ref):
    i = pl.program_id(0)
    h = (raw_ref[...] + g_ref[...]) * dinv_ref[...] + b_ref[...]
    h_ref[...] = h
    ps = jnp.sum(h, 0, keepdims=True)
    pq = jnp.sum(h * h, 0, keepdims=True)
    blk = jnp.concatenate([ps, pq], 0)

    @pl.when(i == 0)
    def _():
        acc_ref[...] = blk

    @pl.when(i > 0)
    def _():
        acc_ref[...] += blk

    @pl.when(i == pl.num_programs(0) - 1)
    def _():
        stat_ref[...] = acc_ref[...]


def _d1(raw2, g2, dinv, b1):
    return pl.pallas_call(
        _d1_body,
        grid=(GRID,),
        in_specs=[
            pl.BlockSpec((BLK, DD), lambda i: (i, 0)),
            pl.BlockSpec((BLK, DD), lambda i: (i, 0)),
            pl.BlockSpec((BLK, 1), lambda i: (i, 0)),
            pl.BlockSpec((1, DD), lambda i: (0, 0)),
        ],
        out_specs=[
            pl.BlockSpec((BLK, DD), lambda i: (i, 0)),
            pl.BlockSpec((2, DD), lambda i: (0, 0)),
        ],
        out_shape=[
            jax.ShapeDtypeStruct((NN, DD), jnp.float32),
            jax.ShapeDtypeStruct((2, DD), jnp.float32),
        ],
        scratch_shapes=[pltpu.VMEM((2, DD), jnp.float32)],
    )(raw2, g2, dinv, b1)


def _d2_body(h_ref, stat_ref, gamma_ref, beta_ref, out_ref):
    mean = stat_ref[0:1] * (1.0 / NN)
    var = stat_ref[1:2] * (1.0 / NN) - mean * mean
    rstd = lax.rsqrt(var + 1e-5)
    out_ref[...] = (h_ref[...] - mean) * rstd * gamma_ref[...] + beta_ref[...]


def _d2(h2, stat, gamma, beta):
    return pl.pallas_call(
        _d2_body,
        grid=(GRID,),
        in_specs=[
            pl.BlockSpec((BLK, DD), lambda i: (i, 0)),
            pl.BlockSpec((2, DD), lambda i: (0, 0)),
            pl.BlockSpec((1, DD), lambda i: (0, 0)),
            pl.BlockSpec((1, DD), lambda i: (0, 0)),
        ],
        out_specs=pl.BlockSpec((BLK, DD), lambda i: (i, 0)),
        out_shape=jax.ShapeDtypeStruct((NN, DD), jnp.float32),
    )(h2, stat, gamma, beta)


def kernel(x, edge_index, W0, b0, W1, b1, gamma, beta):
    ei = edge_index.astype(jnp.int32)
    src, dst = ei[0], ei[1]
    ones = jnp.ones((CH, 16), jnp.float32)
    z1 = jnp.zeros((3128, 16), jnp.float32)
    zrows = jnp.zeros((98, DD), jnp.float32)

    deg2 = _deg_kernel(dst, ones, z1)
    ps, pd, pc = _part_kernel(src, dst)
    g1, dinv = _b1(x, W0, deg2)
    raw1 = _agg(g1, ps, pd, pc, zrows)
    g2 = _b2(raw1, g1, dinv, W1, b0.reshape(1, DD))
    raw2 = _agg(g2, ps, pd, pc, zrows)
    h2, stat = _d1(raw2, g2, dinv, b1.reshape(1, DD))
    out = _d2(h2, stat, gamma.reshape(1, DD), beta.reshape(1, DD))
    return (out[:NUSERS], out[NUSERS:])
